# Initial kernel scaffold; baseline (speedup 1.0000x reference)
import jax
import jax.numpy as jnp
from jax.experimental import pallas as pl


def kernel(h, e, edge_index, eigvecs, eigvals, params):
    raise NotImplementedError("write your pallas kernel here")


# speedup vs baseline: 8.3914x; 8.3914x over previous
import jax
import jax.numpy as jnp
import numpy as np
from jax import lax
from jax.experimental import pallas as pl
from jax.experimental.pallas import tpu as pltpu

N = 10000
E_EDGES = 320000
HIDDEN = 128
HEADS = 8
DH = HIDDEN // HEADS
LPE_DIM = 16
LPE_HEADS = 4
M_EIG = 16
NUM_ATOM = 28

BN_LPE = 200
BN = 2000
BE = 2000


def _ln(x, g, b):
    mu = jnp.mean(x, axis=-1, keepdims=True)
    var = jnp.mean((x - mu) ** 2, axis=-1, keepdims=True)
    return (x - mu) * jax.lax.rsqrt(var + 1e-5) * g + b


def _lpe_body(h_ref, ev_ref, el_ref, emb_h_ref, aw_ref, ab_ref, lpe_refs, out_ref):
    b = BN_LPE
    hv = h_ref[...]
    onehot = (hv == lax.broadcasted_iota(jnp.int32, (b, NUM_ATOM), 1)).astype(jnp.float32)
    emb = jnp.dot(onehot, emb_h_ref[...], preferred_element_type=jnp.float32)

    ev = ev_ref[...]
    el = el_ref[...]
    aw = aw_ref[...]
    pe = (ev[:, :, None] * aw[0][None, None, :]
          + el[:, :, None] * aw[1][None, None, :]
          + ab_ref[...][None, None, :])

    for lyr in lpe_refs:
        x2 = pe.reshape(b * M_EIG, LPE_DIM)
        q = (jnp.dot(x2, lyr['Wq'][...], preferred_element_type=jnp.float32)
             + lyr['bq'][...]).reshape(b, M_EIG, LPE_DIM)
        k = (jnp.dot(x2, lyr['Wk'][...], preferred_element_type=jnp.float32)
             + lyr['bk'][...]).reshape(b, M_EIG, LPE_DIM)
        v = (jnp.dot(x2, lyr['Wv'][...], preferred_element_type=jnp.float32)
             + lyr['bv'][...]).reshape(b, M_EIG, LPE_DIM)
        dh = LPE_DIM // LPE_HEADS
        kt = jnp.swapaxes(k, 1, 2)
        o_cols = [None] * LPE_DIM
        for hh in range(LPE_HEADS):
            att = jnp.zeros((b, M_EIG, M_EIG), jnp.float32)
            for d in range(dh):
                c = hh * dh + d
                att = att + q[:, :, c][:, :, None] * kt[:, c, :][:, None, :]
            att = att * (1.0 / np.sqrt(dh))
            att = att - jnp.max(att, axis=-1, keepdims=True)
            att = jnp.exp(att)
            att = att / jnp.sum(att, axis=-1, keepdims=True)
            for d in range(dh):
                c = hh * dh + d
                o_cols[c] = jnp.sum(att * v[:, :, c][:, None, :], axis=-1)
        o = jnp.concatenate([oc[:, :, None] for oc in o_cols], axis=-1)
        o = o.reshape(b * M_EIG, LPE_DIM)
        o = jnp.dot(o, lyr['Wo'][...], preferred_element_type=jnp.float32) + lyr['bo'][...]
        pe = _ln(pe + o.reshape(b, M_EIG, LPE_DIM), lyr['ln1_g'][...], lyr['ln1_b'][...])
        x2 = pe.reshape(b * M_EIG, LPE_DIM)
        ff = jnp.maximum(
            jnp.dot(x2, lyr['ff1_W'][...], preferred_element_type=jnp.float32)
            + lyr['ff1_b'][...], 0.0)
        ff = jnp.dot(ff, lyr['ff2_W'][...], preferred_element_type=jnp.float32) + lyr['ff2_b'][...]
        pe = _ln(pe + ff.reshape(b, M_EIG, LPE_DIM), lyr['ln2_g'][...], lyr['ln2_b'][...])

    pe_sum = jnp.sum(pe, axis=1)
    out_ref[:, :HIDDEN - LPE_DIM] = emb
    out_ref[:, HIDDEN - LPE_DIM:] = pe_sum


def _lpe_hn(h, eigvecs, eigvals, params):
    lpe = params['lpe']
    lpe_names = ['Wq', 'bq', 'Wk', 'bk', 'Wv', 'bv', 'Wo', 'bo',
                 'ff1_W', 'ff1_b', 'ff2_W', 'ff2_b',
                 'ln1_g', 'ln1_b', 'ln2_g', 'ln2_b']
    flat_w = []
    for lyr in lpe:
        for nm in lpe_names:
            flat_w.append(lyr[nm])

    def body(h_ref, ev_ref, el_ref, emb_ref, aw_ref, ab_ref, *wrefs_and_out):
        wrefs = wrefs_and_out[:-1]
        out_ref = wrefs_and_out[-1]
        lpe_refs = []
        idx = 0
        for _ in lpe:
            d = {}
            for nm in lpe_names:
                d[nm] = wrefs[idx]
                idx += 1
            lpe_refs.append(d)
        _lpe_body(h_ref, ev_ref, el_ref, emb_ref, aw_ref, ab_ref, lpe_refs, out_ref)

    grid = N // BN_LPE
    in_specs = [
        pl.BlockSpec((BN_LPE, 1), lambda i: (i, 0)),
        pl.BlockSpec((BN_LPE, M_EIG), lambda i: (i, 0)),
        pl.BlockSpec((BN_LPE, M_EIG), lambda i: (i, 0)),
    ] + [pl.BlockSpec(w.shape, lambda i, _r=len(w.shape): (0,) * _r)
         for w in [params['emb_h'], params['lpe_A_W'], params['lpe_A_b']] + flat_w]
    return pl.pallas_call(
        body,
        grid=(grid,),
        in_specs=in_specs,
        out_specs=pl.BlockSpec((BN_LPE, HIDDEN), lambda i: (i, 0)),
        out_shape=jax.ShapeDtypeStruct((N, HIDDEN), jnp.float32),
    )(h.reshape(N, 1).astype(jnp.int32), eigvecs, eigvals,
      params['emb_h'], params['lpe_A_W'], params['lpe_A_b'], *flat_w)


def _edge_embed(e, emb_e):
    nb = emb_e.shape[0]

    def body(e_ref, t_ref, o_ref):
        onehot = (e_ref[...] == lax.broadcasted_iota(jnp.int32, (BE, nb), 1)
                  ).astype(jnp.float32)
        o_ref[...] = jnp.dot(onehot, t_ref[...], preferred_element_type=jnp.float32)

    return pl.pallas_call(
        body,
        grid=(E_EDGES // BE,),
        in_specs=[pl.BlockSpec((BE, 1), lambda i: (i, 0)),
                  pl.BlockSpec((nb, HIDDEN), lambda i: (0, 0))],
        out_specs=pl.BlockSpec((BE, HIDDEN), lambda i: (i, 0)),
        out_shape=jax.ShapeDtypeStruct((E_EDGES, HIDDEN), jnp.float32),
    )(e.reshape(E_EDGES, 1).astype(jnp.int32), emb_e)


def _proj(hn, p):
    def body(h_ref, q_ref, k_ref, v_ref, oq, ok, ov):
        x = h_ref[...]
        oq[...] = jnp.dot(x, q_ref[...], preferred_element_type=jnp.float32)
        ok[...] = jnp.dot(x, k_ref[...], preferred_element_type=jnp.float32)
        ov[...] = jnp.dot(x, v_ref[...], preferred_element_type=jnp.float32)

    out = jax.ShapeDtypeStruct((N, HIDDEN), jnp.float32)
    return pl.pallas_call(
        body,
        grid=(N // BN,),
        in_specs=[pl.BlockSpec((BN, HIDDEN), lambda i: (i, 0))]
        + [pl.BlockSpec((HIDDEN, HIDDEN), lambda i: (0, 0))] * 3,
        out_specs=[pl.BlockSpec((BN, HIDDEN), lambda i: (i, 0))] * 3,
        out_shape=[out, out, out],
    )(hn, p['Q'], p['K'], p['V'])


def _edge_layer(en, s0, p):
    def body(en_ref, s0_ref, epr, oew, oeb, f1w, f1b, f2w, f2b,
             l1g, l1b, l2g, l2b, en_out, w_out):
        en_blk = en_ref[...]
        ee = jnp.dot(en_blk, epr[...], preferred_element_type=jnp.float32)
        score = s0_ref[...] * ee * (1.0 / np.sqrt(DH))
        logit = jnp.sum(score.reshape(BE, HEADS, DH), axis=-1)
        w = jnp.exp(jnp.clip(logit, -5.0, 5.0))
        w_out[...] = w
        e_new = jnp.dot(score, oew[...], preferred_element_type=jnp.float32) + oeb[...]
        en1 = _ln(en_blk + e_new, l1g[...], l1b[...])
        ff = jnp.maximum(jnp.dot(en1, f1w[...], preferred_element_type=jnp.float32)
                         + f1b[...], 0.0)
        ff = jnp.dot(ff, f2w[...], preferred_element_type=jnp.float32) + f2b[...]
        en_out[...] = _ln(en1 + ff, l2g[...], l2b[...])

    ws = [p['Epr'], p['Oe_W'], p['Oe_b'], p['ffe1_W'], p['ffe1_b'],
          p['ffe2_W'], p['ffe2_b'], p['ln1e_g'], p['ln1e_b'],
          p['ln2e_g'], p['ln2e_b']]
    return pl.pallas_call(
        body,
        grid=(E_EDGES // BE,),
        in_specs=[pl.BlockSpec((BE, HIDDEN), lambda i: (i, 0)),
                  pl.BlockSpec((BE, HIDDEN), lambda i: (i, 0))]
        + [pl.BlockSpec(w.shape, lambda i, _r=len(w.shape): (0,) * _r) for w in ws],
        out_specs=[pl.BlockSpec((BE, HIDDEN), lambda i: (i, 0)),
                   pl.BlockSpec((BE, HEADS), lambda i: (i, 0))],
        out_shape=[jax.ShapeDtypeStruct((E_EDGES, HIDDEN), jnp.float32),
                   jax.ShapeDtypeStruct((E_EDGES, HEADS), jnp.float32)],
    )(en, s0, *ws)


def _node_layer(hn, wv, z, p):
    def body(h_ref, wv_ref, z_ref, ohw, ohb, f1w, f1b, f2w, f2b,
             l1g, l1b, l2g, l2b, out_ref):
        hn_blk = h_ref[...]
        zinv = 1.0 / (z_ref[...] + 1e-6)
        h_att = (wv_ref[...].reshape(BN, HEADS, DH) * zinv[:, :, None]).reshape(BN, HIDDEN)
        h_new = jnp.dot(h_att, ohw[...], preferred_element_type=jnp.float32) + ohb[...]
        hn1 = _ln(hn_blk + h_new, l1g[...], l1b[...])
        ff = jnp.maximum(jnp.dot(hn1, f1w[...], preferred_element_type=jnp.float32)
                         + f1b[...], 0.0)
        ff = jnp.dot(ff, f2w[...], preferred_element_type=jnp.float32) + f2b[...]
        out_ref[...] = _ln(hn1 + ff, l2g[...], l2b[...])

    ws = [p['Oh_W'], p['Oh_b'], p['ffh1_W'], p['ffh1_b'], p['ffh2_W'], p['ffh2_b'],
          p['ln1h_g'], p['ln1h_b'], p['ln2h_g'], p['ln2h_b']]
    return pl.pallas_call(
        body,
        grid=(N // BN,),
        in_specs=[pl.BlockSpec((BN, HIDDEN), lambda i: (i, 0)),
                  pl.BlockSpec((BN, HIDDEN), lambda i: (i, 0)),
                  pl.BlockSpec((BN, HEADS), lambda i: (i, 0))]
        + [pl.BlockSpec(w.shape, lambda i, _r=len(w.shape): (0,) * _r) for w in ws],
        out_specs=pl.BlockSpec((BN, HIDDEN), lambda i: (i, 0)),
        out_shape=jax.ShapeDtypeStruct((N, HIDDEN), jnp.float32),
    )(hn, wv, z, *ws)


def _readout(hn, mlp):
    def body(h_ref, w1, b1, w2, b2, w3, b3, out_ref):
        hg = jnp.sum(h_ref[...], axis=0, keepdims=True) * (1.0 / N)
        x = jnp.maximum(jnp.dot(hg, w1[...], preferred_element_type=jnp.float32) + b1[...], 0.0)
        x = jnp.maximum(jnp.dot(x, w2[...], preferred_element_type=jnp.float32) + b2[...], 0.0)
        out_ref[...] = jnp.dot(x, w3[...], preferred_element_type=jnp.float32) + b3[...]

    ws = [mlp['W1'], mlp['b1'], mlp['W2'], mlp['b2'], mlp['W3'], mlp['b3']]
    return pl.pallas_call(
        body,
        in_specs=[pl.BlockSpec((N, HIDDEN), lambda: (0, 0))]
        + [pl.BlockSpec(w.shape, lambda _r=len(w.shape): (0,) * _r) for w in ws],
        out_specs=pl.BlockSpec((1, 1), lambda: (0, 0)),
        out_shape=jax.ShapeDtypeStruct((1, 1), jnp.float32),
    )(hn, *ws)


def kernel(h, e, edge_index, eigvecs, eigvals, params):
    hn = _lpe_hn(h, eigvecs, eigvals, params)
    en = _edge_embed(e, params['emb_e'])
    src = edge_index[0]
    dst = edge_index[1]
    for p in params['gt']:
        qh, kh, vh = _proj(hn, p)
        s0 = kh[src] * qh[dst]
        en, w = _edge_layer(en, s0, p)
        wexp = jnp.repeat(w, DH, axis=1)
        wv = jax.ops.segment_sum(wexp * vh[src], dst, num_segments=N)
        z = jax.ops.segment_sum(w, dst, num_segments=N)
        hn = _node_layer(hn, wv, z, p)
    return _readout(hn, params['mlp'])


# speedup vs baseline: 13.6037x; 1.6211x over previous
import functools

import jax
import jax.numpy as jnp
import numpy as np
from jax import lax
from jax.experimental import pallas as pl
from jax.experimental.pallas import tpu as pltpu
from jax.experimental.pallas import tpu_sc as plsc

N = 10000
E_EDGES = 320000
HIDDEN = 128
HEADS = 8
DH = HIDDEN // HEADS
LPE_DIM = 16
LPE_HEADS = 4
M_EIG = 16
NUM_ATOM = 28

BN_LPE = 200
BN = 2000
BE = 2000


def _ln(x, g, b):
    mu = jnp.mean(x, axis=-1, keepdims=True)
    var = jnp.mean((x - mu) ** 2, axis=-1, keepdims=True)
    return (x - mu) * jax.lax.rsqrt(var + 1e-5) * g + b


def _lpe_body(h_ref, ev_ref, el_ref, emb_h_ref, aw_ref, ab_ref, lpe_refs, out_ref):
    b = BN_LPE
    hv = h_ref[...]
    onehot = (hv == lax.broadcasted_iota(jnp.int32, (b, NUM_ATOM), 1)).astype(jnp.float32)
    emb = jnp.dot(onehot, emb_h_ref[...], preferred_element_type=jnp.float32)

    ev = ev_ref[...]
    el = el_ref[...]
    aw = aw_ref[...]
    pe = (ev[:, :, None] * aw[0][None, None, :]
          + el[:, :, None] * aw[1][None, None, :]
          + ab_ref[...][None, None, :])

    for lyr in lpe_refs:
        x2 = pe.reshape(b * M_EIG, LPE_DIM)
        q = (jnp.dot(x2, lyr['Wq'][...], preferred_element_type=jnp.float32)
             + lyr['bq'][...]).reshape(b, M_EIG, LPE_DIM)
        k = (jnp.dot(x2, lyr['Wk'][...], preferred_element_type=jnp.float32)
             + lyr['bk'][...]).reshape(b, M_EIG, LPE_DIM)
        v = (jnp.dot(x2, lyr['Wv'][...], preferred_element_type=jnp.float32)
             + lyr['bv'][...]).reshape(b, M_EIG, LPE_DIM)
        dh = LPE_DIM // LPE_HEADS
        kt = jnp.swapaxes(k, 1, 2)
        o_cols = [None] * LPE_DIM
        for hh in range(LPE_HEADS):
            att = jnp.zeros((b, M_EIG, M_EIG), jnp.float32)
            for d in range(dh):
                c = hh * dh + d
                att = att + q[:, :, c][:, :, None] * kt[:, c, :][:, None, :]
            att = att * (1.0 / np.sqrt(dh))
            att = att - jnp.max(att, axis=-1, keepdims=True)
            att = jnp.exp(att)
            att = att / jnp.sum(att, axis=-1, keepdims=True)
            for d in range(dh):
                c = hh * dh + d
                o_cols[c] = jnp.sum(att * v[:, :, c][:, None, :], axis=-1)
        o = jnp.concatenate([oc[:, :, None] for oc in o_cols], axis=-1)
        o = o.reshape(b * M_EIG, LPE_DIM)
        o = jnp.dot(o, lyr['Wo'][...], preferred_element_type=jnp.float32) + lyr['bo'][...]
        pe = _ln(pe + o.reshape(b, M_EIG, LPE_DIM), lyr['ln1_g'][...], lyr['ln1_b'][...])
        x2 = pe.reshape(b * M_EIG, LPE_DIM)
        ff = jnp.maximum(
            jnp.dot(x2, lyr['ff1_W'][...], preferred_element_type=jnp.float32)
            + lyr['ff1_b'][...], 0.0)
        ff = jnp.dot(ff, lyr['ff2_W'][...], preferred_element_type=jnp.float32) + lyr['ff2_b'][...]
        pe = _ln(pe + ff.reshape(b, M_EIG, LPE_DIM), lyr['ln2_g'][...], lyr['ln2_b'][...])

    pe_sum = jnp.sum(pe, axis=1)
    out_ref[:, :HIDDEN - LPE_DIM] = emb
    out_ref[:, HIDDEN - LPE_DIM:] = pe_sum


def _lpe_hn(h, eigvecs, eigvals, params):
    lpe = params['lpe']
    lpe_names = ['Wq', 'bq', 'Wk', 'bk', 'Wv', 'bv', 'Wo', 'bo',
                 'ff1_W', 'ff1_b', 'ff2_W', 'ff2_b',
                 'ln1_g', 'ln1_b', 'ln2_g', 'ln2_b']
    flat_w = []
    for lyr in lpe:
        for nm in lpe_names:
            flat_w.append(lyr[nm])

    def body(h_ref, ev_ref, el_ref, emb_ref, aw_ref, ab_ref, *wrefs_and_out):
        wrefs = wrefs_and_out[:-1]
        out_ref = wrefs_and_out[-1]
        lpe_refs = []
        idx = 0
        for _ in lpe:
            d = {}
            for nm in lpe_names:
                d[nm] = wrefs[idx]
                idx += 1
            lpe_refs.append(d)
        _lpe_body(h_ref, ev_ref, el_ref, emb_ref, aw_ref, ab_ref, lpe_refs, out_ref)

    grid = N // BN_LPE
    in_specs = [
        pl.BlockSpec((BN_LPE, 1), lambda i: (i, 0)),
        pl.BlockSpec((BN_LPE, M_EIG), lambda i: (i, 0)),
        pl.BlockSpec((BN_LPE, M_EIG), lambda i: (i, 0)),
    ] + [pl.BlockSpec(w.shape, lambda i, _r=len(w.shape): (0,) * _r)
         for w in [params['emb_h'], params['lpe_A_W'], params['lpe_A_b']] + flat_w]
    return pl.pallas_call(
        body,
        grid=(grid,),
        in_specs=in_specs,
        out_specs=pl.BlockSpec((BN_LPE, HIDDEN), lambda i: (i, 0)),
        out_shape=jax.ShapeDtypeStruct((N, HIDDEN), jnp.float32),
    )(h.reshape(N, 1).astype(jnp.int32), eigvecs, eigvals,
      params['emb_h'], params['lpe_A_W'], params['lpe_A_b'], *flat_w)


def _edge_embed(e, emb_e):
    nb = emb_e.shape[0]

    def body(e_ref, t_ref, o_ref):
        onehot = (e_ref[...] == lax.broadcasted_iota(jnp.int32, (BE, nb), 1)
                  ).astype(jnp.float32)
        o_ref[...] = jnp.dot(onehot, t_ref[...], preferred_element_type=jnp.float32)

    return pl.pallas_call(
        body,
        grid=(E_EDGES // BE,),
        in_specs=[pl.BlockSpec((BE, 1), lambda i: (i, 0)),
                  pl.BlockSpec((nb, HIDDEN), lambda i: (0, 0))],
        out_specs=pl.BlockSpec((BE, HIDDEN), lambda i: (i, 0)),
        out_shape=jax.ShapeDtypeStruct((E_EDGES, HIDDEN), jnp.float32),
    )(e.reshape(E_EDGES, 1).astype(jnp.int32), emb_e)


def _proj(hn, p):
    def body(h_ref, q_ref, k_ref, v_ref, oq, ok, ov):
        x = h_ref[...]
        oq[...] = jnp.dot(x, q_ref[...], preferred_element_type=jnp.float32)
        ok[...] = jnp.dot(x, k_ref[...], preferred_element_type=jnp.float32)
        ov[...] = jnp.dot(x, v_ref[...], preferred_element_type=jnp.float32)

    out = jax.ShapeDtypeStruct((N, HIDDEN), jnp.float32)
    return pl.pallas_call(
        body,
        grid=(N // BN,),
        in_specs=[pl.BlockSpec((BN, HIDDEN), lambda i: (i, 0))]
        + [pl.BlockSpec((HIDDEN, HIDDEN), lambda i: (0, 0))] * 3,
        out_specs=[pl.BlockSpec((BN, HIDDEN), lambda i: (i, 0))] * 3,
        out_shape=[out, out, out],
    )(hn, p['Q'], p['K'], p['V'])


def _edge_layer(en, s0, p):
    def body(en_ref, s0_ref, epr, oew, oeb, f1w, f1b, f2w, f2b,
             l1g, l1b, l2g, l2b, en_out, w_out):
        en_blk = en_ref[...]
        ee = jnp.dot(en_blk, epr[...], preferred_element_type=jnp.float32)
        score = s0_ref[...] * ee * (1.0 / np.sqrt(DH))
        logit = jnp.sum(score.reshape(BE, HEADS, DH), axis=-1)
        w = jnp.exp(jnp.clip(logit, -5.0, 5.0))
        w_out[...] = jnp.concatenate([w, jnp.zeros((BE, 8), jnp.float32)], axis=-1)
        e_new = jnp.dot(score, oew[...], preferred_element_type=jnp.float32) + oeb[...]
        en1 = _ln(en_blk + e_new, l1g[...], l1b[...])
        ff = jnp.maximum(jnp.dot(en1, f1w[...], preferred_element_type=jnp.float32)
                         + f1b[...], 0.0)
        ff = jnp.dot(ff, f2w[...], preferred_element_type=jnp.float32) + f2b[...]
        en_out[...] = _ln(en1 + ff, l2g[...], l2b[...])

    ws = [p['Epr'], p['Oe_W'], p['Oe_b'], p['ffe1_W'], p['ffe1_b'],
          p['ffe2_W'], p['ffe2_b'], p['ln1e_g'], p['ln1e_b'],
          p['ln2e_g'], p['ln2e_b']]
    return pl.pallas_call(
        body,
        grid=(E_EDGES // BE,),
        in_specs=[pl.BlockSpec((BE, HIDDEN), lambda i: (i, 0)),
                  pl.BlockSpec((BE, HIDDEN), lambda i: (i, 0))]
        + [pl.BlockSpec(w.shape, lambda i, _r=len(w.shape): (0,) * _r) for w in ws],
        out_specs=[pl.BlockSpec((BE, HIDDEN), lambda i: (i, 0)),
                   pl.BlockSpec((BE, 16), lambda i: (i, 0))],
        out_shape=[jax.ShapeDtypeStruct((E_EDGES, HIDDEN), jnp.float32),
                   jax.ShapeDtypeStruct((E_EDGES, 16), jnp.float32)],
    )(en, s0, *ws)


def _node_layer(hn, wv, z, p):
    def body(h_ref, wv_ref, z_ref, ohw, ohb, f1w, f1b, f2w, f2b,
             l1g, l1b, l2g, l2b, out_ref):
        hn_blk = h_ref[...]
        zs = z_ref[0] + z_ref[1]
        zinv = 1.0 / (zs[:, :HEADS] + 1e-6)
        wvs = wv_ref[0] + wv_ref[1]
        h_att = (wvs.reshape(BN, HEADS, DH) * zinv[:, :, None]).reshape(BN, HIDDEN)
        h_new = jnp.dot(h_att, ohw[...], preferred_element_type=jnp.float32) + ohb[...]
        hn1 = _ln(hn_blk + h_new, l1g[...], l1b[...])
        ff = jnp.maximum(jnp.dot(hn1, f1w[...], preferred_element_type=jnp.float32)
                         + f1b[...], 0.0)
        ff = jnp.dot(ff, f2w[...], preferred_element_type=jnp.float32) + f2b[...]
        out_ref[...] = _ln(hn1 + ff, l2g[...], l2b[...])

    ws = [p['Oh_W'], p['Oh_b'], p['ffh1_W'], p['ffh1_b'], p['ffh2_W'], p['ffh2_b'],
          p['ln1h_g'], p['ln1h_b'], p['ln2h_g'], p['ln2h_b']]
    return pl.pallas_call(
        body,
        grid=(N // BN,),
        in_specs=[pl.BlockSpec((BN, HIDDEN), lambda i: (i, 0)),
                  pl.BlockSpec((2, BN, HIDDEN), lambda i: (0, i, 0)),
                  pl.BlockSpec((2, BN, 16), lambda i: (0, i, 0))]
        + [pl.BlockSpec(w.shape, lambda i, _r=len(w.shape): (0,) * _r) for w in ws],
        out_specs=pl.BlockSpec((BN, HIDDEN), lambda i: (i, 0)),
        out_shape=jax.ShapeDtypeStruct((N, HIDDEN), jnp.float32),
    )(hn, wv, z, *ws)


def _readout(hn, mlp):
    def body(h_ref, w1, b1, w2, b2, w3, b3, out_ref):
        hg = jnp.sum(h_ref[...], axis=0, keepdims=True) * (1.0 / N)
        x = jnp.maximum(jnp.dot(hg, w1[...], preferred_element_type=jnp.float32) + b1[...], 0.0)
        x = jnp.maximum(jnp.dot(x, w2[...], preferred_element_type=jnp.float32) + b2[...], 0.0)
        out_ref[...] = jnp.dot(x, w3[...], preferred_element_type=jnp.float32) + b3[...]

    ws = [mlp['W1'], mlp['b1'], mlp['W2'], mlp['b2'], mlp['W3'], mlp['b3']]
    return pl.pallas_call(
        body,
        in_specs=[pl.BlockSpec((N, HIDDEN), lambda: (0, 0))]
        + [pl.BlockSpec(w.shape, lambda _r=len(w.shape): (0,) * _r) for w in ws],
        out_specs=pl.BlockSpec((1, 1), lambda: (0, 0)),
        out_shape=jax.ShapeDtypeStruct((1, 1), jnp.float32),
    )(hn, *ws)


_NC, _NS = 2, 16
_NW = _NC * _NS
_EW = E_EDGES // _NW
_CSC = 80
_CSA = 400
N_PAD = 10240
_NROW = N_PAD // _NS


def _s0_gather(kh, qh, src, dst):
    mesh = plsc.VectorSubcoreMesh(core_axis_name="c", subcore_axis_name="s")

    @functools.partial(
        pl.kernel,
        out_type=jax.ShapeDtypeStruct((E_EDGES, HIDDEN), jnp.float32),
        mesh=mesh,
        scratch_types=[
            pltpu.VMEM((_CSA,), jnp.int32),
            pltpu.VMEM((_CSA,), jnp.int32),
            pltpu.VMEM((_CSA, HIDDEN), jnp.float32),
            pltpu.VMEM((_CSA, HIDDEN), jnp.float32),
            pltpu.SemaphoreType.DMA,
        ],
    )
    def k(kh_hbm, qh_hbm, src_hbm, dst_hbm, out_hbm, sidx, didx, krows, qrows, sem):
        wid = lax.axis_index("s") * _NC + lax.axis_index("c")
        base = wid * _EW

        def step(i, _):
            off = base + i * _CSA
            pltpu.sync_copy(src_hbm.at[pl.ds(off, _CSA)], sidx)
            pltpu.sync_copy(dst_hbm.at[pl.ds(off, _CSA)], didx)
            ck = pltpu.async_copy(kh_hbm.at[sidx], krows, sem)
            cq = pltpu.async_copy(qh_hbm.at[didx], qrows, sem)
            ck.wait()
            cq.wait()

            def mul_e(ei, _):
                for d in range(HEADS):
                    sl = pl.ds(d * 16, 16)
                    krows[ei, sl] = krows[ei, sl] * qrows[ei, sl]
                return 0

            lax.fori_loop(0, _CSA, mul_e, 0)
            pltpu.sync_copy(krows, out_hbm.at[pl.ds(off, _CSA)])
            return 0

        lax.fori_loop(0, _EW // _CSA, step, 0)

    return k(kh, qh, src, dst)


_NZP = N_PAD // 8
_NZROW = _NZP // _NS


def _wv_scatter(vh, src, dst, dst8, dstg, w, zeros_wv):
    mesh = plsc.VectorSubcoreMesh(core_axis_name="c", subcore_axis_name="s")

    @functools.partial(
        pl.kernel,
        out_type=(jax.ShapeDtypeStruct((_NC, N_PAD, HIDDEN), jnp.float32),
                  jax.ShapeDtypeStruct((_NC, _NZP, HIDDEN), jnp.float32)),
        mesh=mesh,
        scratch_types=[
            pltpu.VMEM((_CSC,), jnp.int32),
            pltpu.VMEM((_CSC,), jnp.int32),
            pltpu.VMEM((_CSC,), jnp.int32),
            pltpu.VMEM((_CSC,), jnp.int32),
            pltpu.VMEM((_CSC, HIDDEN), jnp.float32),
            pltpu.VMEM((_CSC, 16), jnp.float32),
            pltpu.VMEM((_CSC, HIDDEN), jnp.float32),
            pltpu.VMEM_SHARED((N_PAD, HIDDEN), jnp.float32),
            pltpu.VMEM_SHARED((_NZP, HIDDEN), jnp.float32),
            pltpu.SemaphoreType.DMA,
        ],
    )
    def k(vh_hbm, src_hbm, dst_hbm, d8_hbm, dg_hbm, w_hbm, zwv_hbm,
          wv_out, zp_out,
          sidx, didx, d8idx, dgidx, vrows, wrows, wxbuf, wv_acc, zp_acc, sem):
        cid = lax.axis_index("c")
        sid = lax.axis_index("s")
        wid = sid * _NC + cid
        base = wid * _EW
        r0 = sid * _NROW
        z0 = sid * _NZROW
        pltpu.sync_copy(zwv_hbm.at[pl.ds(r0, _NROW)], wv_acc.at[pl.ds(r0, _NROW)])
        pltpu.sync_copy(zwv_hbm.at[pl.ds(z0, _NZROW)], zp_acc.at[pl.ds(z0, _NZROW)])
        plsc.subcore_barrier()

        dnums = lax.GatherDimensionNumbers(
            offset_dims=(), collapsed_slice_dims=(0,), start_index_map=(0,))
        z16 = jnp.zeros((16,), jnp.float32)

        def step(i, _):
            off = base + i * _CSC
            pltpu.sync_copy(src_hbm.at[pl.ds(off, _CSC)], sidx)
            pltpu.sync_copy(dst_hbm.at[pl.ds(off, _CSC)], didx)
            pltpu.sync_copy(d8_hbm.at[pl.ds(off, _CSC)], d8idx)
            pltpu.sync_copy(dg_hbm.at[pl.ds(off, _CSC)], dgidx)
            pltpu.sync_copy(w_hbm.at[pl.ds(off, _CSC)], wrows)
            pltpu.async_copy(vh_hbm.at[sidx], vrows, sem).wait()

            def mul_j(j, _):
                gv = dgidx[pl.ds(j * 16, 16)]
                for l in range(16):
                    ei = j * 16 + l
                    wr = wrows[ei, :]
                    for d in range(HEADS):
                        sl = pl.ds(d * 16, 16)
                        ws = lax.gather(
                            wr, jnp.full((16, 1), d, jnp.int32), dnums, (1,),
                            mode=lax.GatherScatterMode.PROMISE_IN_BOUNDS)
                        vrows[ei, sl] = vrows[ei, sl] * ws
                    ges = lax.gather(
                        gv, jnp.full((16, 1), l, jnp.int32), dnums, (1,),
                        mode=lax.GatherScatterMode.PROMISE_IN_BOUNDS)
                    for g in range(8):
                        sl = pl.ds(g * 16, 16)
                        ind = (1 - jnp.minimum(jnp.abs(ges - g), 1)).astype(jnp.float32)
                        wxbuf[ei, sl] = wr * ind
                return 0

            lax.fori_loop(0, _CSC // 16, mul_j, 0)
            pltpu.sync_copy(vrows, wv_acc.at[didx], add=True)
            pltpu.sync_copy(wxbuf, zp_acc.at[d8idx], add=True)
            return 0

        lax.fori_loop(0, _EW // _CSC, step, 0)
        plsc.subcore_barrier()
        pltpu.sync_copy(wv_acc.at[pl.ds(r0, _NROW)], wv_out.at[cid, pl.ds(r0, _NROW)])
        pltpu.sync_copy(zp_acc.at[pl.ds(z0, _NZROW)], zp_out.at[cid, pl.ds(z0, _NZROW)])

    return k(vh, src, dst, dst8, dstg, w, zeros_wv)


def kernel(h, e, edge_index, eigvecs, eigvals, params):
    hn = _lpe_hn(h, eigvecs, eigvals, params)
    en = _edge_embed(e, params['emb_e'])
    src = edge_index[0].astype(jnp.int32)
    dst = edge_index[1].astype(jnp.int32)
    dst8 = dst >> 3
    dstg = dst & 7
    zeros_wv = jnp.zeros((N_PAD, HIDDEN), jnp.float32)
    for p in params['gt']:
        qh, kh, vh = _proj(hn, p)
        s0 = _s0_gather(kh, qh, src, dst)
        en, w = _edge_layer(en, s0, p)
        wv2, zp2 = _wv_scatter(vh, src, dst, dst8, dstg, w, zeros_wv)
        z2 = zp2.reshape(_NC, N_PAD, 16)
        hn = _node_layer(hn, wv2, z2, p)
    return _readout(hn, params['mlp'])
